# Initial kernel scaffold; baseline (speedup 1.0000x reference)
import jax
import jax.numpy as jnp
from jax.experimental import pallas as pl


def kernel(prefix, time_vector, embedding):
    raise NotImplementedError("write your pallas kernel here")


# speedup vs baseline: 1.1578x; 1.1578x over previous
import functools

import jax
import jax.numpy as jnp
from jax import lax
from jax.experimental import pallas as pl
from jax.experimental.pallas import tpu as pltpu
from jax.experimental.pallas import tpu_sc as plsc

NC = 2
NS = 16
NW = NC * NS
LANES = 16


def _sc_lookup_add(idx, tv, emb, *, chunk):
    n, d = tv.shape
    n_per_w = n // NW
    n_chunks = n_per_w // chunk
    mesh = plsc.VectorSubcoreMesh(core_axis_name="c", subcore_axis_name="s")

    @functools.partial(
        pl.kernel,
        mesh=mesh,
        out_type=jax.ShapeDtypeStruct((n, d), jnp.float32),
        scratch_types=[
            pltpu.VMEM((n_per_w,), jnp.int32),
            pltpu.VMEM((chunk, d), jnp.float32),
            pltpu.VMEM((chunk, d), jnp.float32),
            pltpu.SemaphoreType.DMA,
        ],
    )
    def k(idx_hbm, tv_hbm, emb_hbm, out_hbm, idx_v, tv_buf, emb_buf, sem):
        wid = lax.axis_index("s") * NC + lax.axis_index("c")
        base = wid * n_per_w
        pltpu.sync_copy(idx_hbm.at[pl.ds(base, n_per_w)], idx_v)

        def chunk_body(c, carry):
            row0 = c * chunk
            gather = pltpu.async_copy(
                emb_hbm.at[idx_v.at[pl.ds(row0, chunk)]], emb_buf, sem
            )
            pltpu.sync_copy(tv_hbm.at[pl.ds(base + row0, chunk)], tv_buf)
            gather.wait()

            def add_body(i, carry2):
                r = i // (d // LANES)
                col = (i % (d // LANES)) * LANES
                sl = pl.ds(col, LANES)
                tv_buf[r, sl] += emb_buf[r, sl]
                return carry2

            lax.fori_loop(0, chunk * (d // LANES), add_body, 0, unroll=4)
            pltpu.sync_copy(tv_buf, out_hbm.at[pl.ds(base + row0, chunk)])
            return carry

        lax.fori_loop(0, n_chunks, chunk_body, 0)

    return k(idx, tv, emb)


def kernel(prefix, time_vector, embedding):
    b, s = prefix.shape
    v, d = embedding.shape
    n = b * s
    idx = prefix.reshape(n).astype(jnp.int32)
    tv = time_vector.reshape(n, d)
    out = _sc_lookup_add(idx, tv, embedding, chunk=40)
    return out.reshape(b, s, d)


# speedup vs baseline: 1.6331x; 1.4104x over previous
import functools

import jax
import jax.numpy as jnp
from jax import lax
from jax.experimental import pallas as pl
from jax.experimental.pallas import tpu as pltpu
from jax.experimental.pallas import tpu_sc as plsc

NC = 2
NS = 16
NW = NC * NS
LANES = 16


def _sc_lookup_add(idx, tv, emb, *, chunk):
    n, d = tv.shape
    n_per_w = n // NW
    n_chunks = n_per_w // chunk
    assert n_chunks % 2 == 0
    mesh = plsc.VectorSubcoreMesh(core_axis_name="c", subcore_axis_name="s")

    @functools.partial(
        pl.kernel,
        mesh=mesh,
        out_type=jax.ShapeDtypeStruct((n, d), jnp.float32),
        scratch_types=[
            pltpu.VMEM((n_per_w,), jnp.int32),
            pltpu.VMEM((chunk, d), jnp.float32),
            pltpu.VMEM((chunk, d), jnp.float32),
            pltpu.VMEM((chunk, d), jnp.float32),
            pltpu.VMEM((chunk, d), jnp.float32),
            pltpu.SemaphoreType.DMA,
            pltpu.SemaphoreType.DMA,
            pltpu.SemaphoreType.DMA,
            pltpu.SemaphoreType.DMA,
            pltpu.SemaphoreType.DMA,
            pltpu.SemaphoreType.DMA,
        ],
    )
    def k(idx_hbm, tv_hbm, emb_hbm, out_hbm, idx_v,
          tv0, tv1, em0, em1, st0, st1, sg0, sg1, so0, so1):
        wid = lax.axis_index("s") * NC + lax.axis_index("c")
        base = wid * n_per_w
        tv_bufs = (tv0, tv1)
        em_bufs = (em0, em1)
        sem_tv = (st0, st1)
        sem_g = (sg0, sg1)
        sem_out = (so0, so1)

        pltpu.sync_copy(idx_hbm.at[pl.ds(base, n_per_w)], idx_v)

        def start_in(c, b):
            row0 = c * chunk
            pltpu.async_copy(
                tv_hbm.at[pl.ds(base + row0, chunk)], tv_bufs[b], sem_tv[b]
            )
            pltpu.async_copy(
                emb_hbm.at[idx_v.at[pl.ds(row0, chunk)]], em_bufs[b], sem_g[b]
            )

        def wait_in(b):
            pltpu.make_async_copy(
                tv_hbm.at[pl.ds(base, chunk)], tv_bufs[b], sem_tv[b]
            ).wait()
            pltpu.make_async_copy(
                emb_hbm.at[idx_v.at[pl.ds(0, chunk)]], em_bufs[b], sem_g[b]
            ).wait()

        def wait_out(b):
            pltpu.make_async_copy(
                tv_bufs[b], out_hbm.at[pl.ds(base, chunk)], sem_out[b]
            ).wait()

        def add_chunk(b):
            def add_body(i, carry):
                r = i // (d // LANES)
                col = (i % (d // LANES)) * LANES
                sl = pl.ds(col, LANES)
                tv_bufs[b][r, sl] += em_bufs[b][r, sl]
                return carry

            lax.fori_loop(0, chunk * (d // LANES), add_body, 0, unroll=4)

        start_in(0, 0)

        def pair_body(i, carry):
            for b in (0, 1):
                c = 2 * i + b
                q = 1 - b
                if b == 0:
                    @pl.when(i > 0)
                    def _():
                        wait_out(q)
                    start_in(c + 1, q)
                else:
                    wait_out(q)

                    @pl.when(i < n_chunks // 2 - 1)
                    def _():
                        start_in(c + 1, q)
                wait_in(b)
                add_chunk(b)
                row0 = c * chunk
                pltpu.async_copy(
                    tv_bufs[b], out_hbm.at[pl.ds(base + row0, chunk)], sem_out[b]
                )
            return carry

        lax.fori_loop(0, n_chunks // 2, pair_body, 0)
        wait_out(1)

    return k(idx, tv, emb)


def kernel(prefix, time_vector, embedding):
    b, s = prefix.shape
    v, d = embedding.shape
    n = b * s
    idx = prefix.reshape(n).astype(jnp.int32)
    tv = time_vector.reshape(n, d)
    out = _sc_lookup_add(idx, tv, embedding, chunk=16)
    return out.reshape(b, s, d)
